# Initial kernel scaffold; baseline (speedup 1.0000x reference)
import jax
import jax.numpy as jnp
from jax.experimental import pallas as pl


def kernel(x, nodes, nbr1, nbr2, W1, W2, Wp, bp):
    raise NotImplementedError("write your pallas kernel here")


# speedup vs baseline: 2.1681x; 2.1681x over previous
import functools

import jax
import jax.numpy as jnp
from jax import lax
from jax.experimental import pallas as pl
from jax.experimental.pallas import tpu as pltpu
from jax.experimental.pallas import tpu_sc as plsc

T = 3
N = 100000
D = 128
B = 10000
S1, S2 = 5, 2
H1, H2 = 512, 10
OUT = 32
VTH = 1.0

NC, NS = 2, 16
NW = NC * NS
BP = 10240

_C0, _K0 = 96, 10
_C1, _K1 = 120, 40
_C2, _K2 = 128, 75

BLK = 512


def _sc_gather_build():
    mesh = plsc.VectorSubcoreMesh(core_axis_name="c", subcore_axis_name="s")

    @functools.partial(
        pl.kernel,
        out_type=[
            jax.ShapeDtypeStruct((T * BP, D), jnp.float32),
            jax.ShapeDtypeStruct((T * BP * S1, D), jnp.float32),
            jax.ShapeDtypeStruct((T * BP * S1 * S2, D), jnp.float32),
        ],
        mesh=mesh,
        scratch_types=[
            pltpu.VMEM((_C0,), jnp.int32),
            pltpu.VMEM((_C1,), jnp.int32),
            pltpu.VMEM((_C2,), jnp.int32),
            pltpu.VMEM((_C0, D), jnp.float32),
            pltpu.VMEM((_C1, D), jnp.float32),
            pltpu.VMEM((_C2, D), jnp.float32),
            pltpu.SemaphoreType.DMA,
        ],
    )
    def sc_gather(table, i0, i1, i2, o0, o1, o2,
                  iv0, iv1, iv2, rv0, rv1, rv2, sem):
        wid = lax.axis_index("s") * NC + lax.axis_index("c")

        def section(idx_hbm, out_hbm, iv, rv, chunk, nchunks):
            base_w = wid * (chunk * nchunks)

            def step(i, carry):
                base = base_w + i * chunk
                pltpu.sync_copy(idx_hbm.at[pl.ds(base, chunk)], iv)
                pltpu.async_copy(table.at[iv], rv, sem).wait()
                pltpu.sync_copy(rv, out_hbm.at[pl.ds(base, chunk)])
                return carry

            lax.fori_loop(0, nchunks, step, 0)

        section(i0, o0, iv0, rv0, _C0, _K0)
        section(i1, o1, iv1, rv1, _C1, _K1)
        section(i2, o2, iv2, rv2, _C2, _K2)

    return sc_gather


_sc_gather = _sc_gather_build()


def _tc_body(h0, h1, h2, w1, w2, wp, bpr, out):
    w1v = w1[...]
    w2v = w2[...]
    acc = jnp.zeros((BLK, OUT), jnp.float32) + bpr[...]
    for t in range(T):
        n0 = h1[t, :, 0, :]
        for k in range(1, S1):
            n0 = n0 + h1[t, :, k, :]
        a0 = h0[t] + n0 * (1.0 / S1)
        g = (jnp.dot(a0, w1v, preferred_element_type=jnp.float32)
             >= VTH).astype(jnp.float32)
        gs = jnp.zeros((BLK, H1), jnp.float32)
        for k in range(S1):
            a1 = h1[t, :, k, :] + 0.5 * (h2[t, :, 2 * k, :]
                                         + h2[t, :, 2 * k + 1, :])
            s1 = (jnp.dot(a1, w1v, preferred_element_type=jnp.float32)
                  >= VTH).astype(jnp.float32)
            gs = gs + s1
        g2 = g + gs * (1.0 / S1)
        o2 = jnp.dot(g2, w2v, preferred_element_type=jnp.float32)
        s2 = (o2 >= VTH).astype(jnp.float32)
        acc = acc + jnp.dot(s2, wp[t * H2:(t + 1) * H2, :],
                            preferred_element_type=jnp.float32)
    out[...] = acc


_tc_net = pl.pallas_call(
    _tc_body,
    grid=(BP // BLK,),
    in_specs=[
        pl.BlockSpec((T, BLK, D), lambda i: (0, i, 0)),
        pl.BlockSpec((T, BLK, S1, D), lambda i: (0, i, 0, 0)),
        pl.BlockSpec((T, BLK, S1 * S2, D), lambda i: (0, i, 0, 0)),
        pl.BlockSpec((D, H1), lambda i: (0, 0)),
        pl.BlockSpec((H1, H2), lambda i: (0, 0)),
        pl.BlockSpec((T * H2, OUT), lambda i: (0, 0)),
        pl.BlockSpec((1, OUT), lambda i: (0, 0)),
    ],
    out_specs=pl.BlockSpec((BLK, OUT), lambda i: (i, 0)),
    out_shape=jax.ShapeDtypeStruct((BP, OUT), jnp.float32),
)


def kernel(x, nodes, nbr1, nbr2, W1, W2, Wp, bp):
    table = x.reshape(T * N, D)
    off = (jnp.arange(T, dtype=jnp.int32) * N)[:, None]
    nodes_p = jnp.pad(nodes.astype(jnp.int32), (0, BP - B))
    idx0 = (nodes_p[None, :] + off).reshape(-1)
    n1 = jnp.pad(nbr1.astype(jnp.int32).reshape(T, B, S1),
                 ((0, 0), (0, BP - B), (0, 0)))
    idx1 = (n1 + off[:, :, None]).reshape(-1)
    n2 = jnp.pad(nbr2.astype(jnp.int32).reshape(T, B, S1 * S2),
                 ((0, 0), (0, BP - B), (0, 0)))
    idx2 = (n2 + off[:, :, None]).reshape(-1)

    h0g, h1g, h2g = _sc_gather(table, idx0, idx1, idx2)

    out = _tc_net(
        h0g.reshape(T, BP, D),
        h1g.reshape(T, BP, S1, D),
        h2g.reshape(T, BP, S1 * S2, D),
        W1, W2, Wp, bp.reshape(1, OUT),
    )
    return out[:B]


# speedup vs baseline: 3.9033x; 1.8003x over previous
import functools

import jax
import jax.numpy as jnp
from jax import lax
from jax.experimental import pallas as pl
from jax.experimental.pallas import tpu as pltpu
from jax.experimental.pallas import tpu_sc as plsc

T = 3
N = 100000
D = 128
B = 10000
S1, S2 = 5, 2
H1, H2 = 512, 10
OUT = 32
VTH = 1.0

NC, NS = 2, 16
NW = NC * NS
ROWS = T * B * (1 + S1 + S1 * S2)
CHUNK = 128
KCH = 118
RPW = CHUNK * KCH
TOT = RPW * NW

BLK = 1000
NB = B // BLK


def _sc_gather_build():
    mesh = plsc.VectorSubcoreMesh(core_axis_name="c", subcore_axis_name="s")

    @functools.partial(
        pl.kernel,
        out_type=jax.ShapeDtypeStruct((TOT, D), jnp.float32),
        mesh=mesh,
        scratch_types=[
            pltpu.VMEM((RPW,), jnp.int32),
            pltpu.VMEM((CHUNK, D), jnp.float32),
            pltpu.SemaphoreType.DMA,
        ],
    )
    def sc_gather(table, idx, out, iv, rv, sem):
        wid = lax.axis_index("s") * NC + lax.axis_index("c")
        base_w = wid * RPW
        pltpu.sync_copy(idx.at[pl.ds(base_w, RPW)], iv)

        def step(i, carry):
            pltpu.async_copy(table.at[iv.at[pl.ds(i * CHUNK, CHUNK)]],
                             rv, sem).wait()
            pltpu.sync_copy(rv, out.at[pl.ds(base_w + i * CHUNK, CHUNK)])
            return carry

        lax.fori_loop(0, KCH, step, 0)

    return sc_gather


_sc_gather = _sc_gather_build()


def _tc_body(h0, h10, h11, h12, h13, h14,
             h20, h21, h22, h23, h24, h25, h26, h27, h28, h29,
             w1, w2, wp, bpr, out):
    h1b = (h10, h11, h12, h13, h14)
    h2b = (h20, h21, h22, h23, h24, h25, h26, h27, h28, h29)
    w1v = w1[...]
    n0 = h1b[0][...]
    for k in range(1, S1):
        n0 = n0 + h1b[k][...]
    a0 = h0[...] + n0 * (1.0 / S1)
    g = (jnp.dot(a0, w1v, preferred_element_type=jnp.float32)
         >= VTH).astype(jnp.float32)
    gs = jnp.zeros((BLK, H1), jnp.float32)
    for k in range(S1):
        a1 = h1b[k][...] + 0.5 * (h2b[2 * k][...] + h2b[2 * k + 1][...])
        s1 = (jnp.dot(a1, w1v, preferred_element_type=jnp.float32)
              >= VTH).astype(jnp.float32)
        gs = gs + s1
    g2 = g + gs * (1.0 / S1)
    o2 = jnp.dot(g2, w2[...], preferred_element_type=jnp.float32)
    s2 = (o2 >= VTH).astype(jnp.float32)
    contrib = jnp.dot(s2, wp[0], preferred_element_type=jnp.float32)
    t = pl.program_id(1)

    @pl.when(t == 0)
    def _():
        out[...] = bpr[...] + contrib

    @pl.when(t > 0)
    def _():
        out[...] = out[...] + contrib


_IN_SPECS = (
    [pl.BlockSpec((BLK, D), lambda i, t: (t * NB + i, 0))]
    + [pl.BlockSpec((BLK, D),
                    lambda i, t, k=k: ((3 + t * S1 + k) * NB + i, 0))
       for k in range(S1)]
    + [pl.BlockSpec((BLK, D),
                    lambda i, t, m=m: ((18 + t * S1 * S2 + m) * NB + i, 0))
       for m in range(S1 * S2)]
    + [
        pl.BlockSpec((D, H1), lambda i, t: (0, 0)),
        pl.BlockSpec((H1, H2), lambda i, t: (0, 0)),
        pl.BlockSpec((1, H2, OUT), lambda i, t: (t, 0, 0)),
        pl.BlockSpec((1, OUT), lambda i, t: (0, 0)),
    ]
)

_tc_net = pl.pallas_call(
    _tc_body,
    grid=(NB, T),
    in_specs=_IN_SPECS,
    out_specs=pl.BlockSpec((BLK, OUT), lambda i, t: (i, 0)),
    out_shape=jax.ShapeDtypeStruct((B, OUT), jnp.float32),
)


def kernel(x, nodes, nbr1, nbr2, W1, W2, Wp, bp):
    table = x.reshape(T * N, D)
    off = jnp.arange(T, dtype=jnp.int32) * N
    idx0 = (off[:, None] + nodes.astype(jnp.int32)[None, :]).reshape(-1)
    i1 = nbr1.astype(jnp.int32).reshape(T, B, S1).transpose(0, 2, 1)
    idx1 = (off[:, None, None] + i1).reshape(-1)
    i2 = nbr2.astype(jnp.int32).reshape(T, B, S1 * S2).transpose(0, 2, 1)
    idx2 = (off[:, None, None] + i2).reshape(-1)
    idx = jnp.concatenate(
        [idx0, idx1, idx2, jnp.zeros((TOT - ROWS,), jnp.int32)])

    hg = _sc_gather(table, idx)

    args = [hg] * 16 + [W1, W2, Wp.reshape(T, H2, OUT), bp.reshape(1, OUT)]
    return _tc_net(*args)


# speedup vs baseline: 6.8826x; 1.7633x over previous
import functools

import numpy as np

import jax
import jax.numpy as jnp
from jax import lax
from jax.experimental import pallas as pl
from jax.experimental.pallas import tpu as pltpu
from jax.experimental.pallas import tpu_sc as plsc

T = 3
N = 100000
D = 128
B = 10000
S1, S2 = 5, 2
H1, H2 = 512, 10
OUT = 32
VTH = 1.0

NC, NS = 2, 16
NW = NC * NS
ROWS_T = B * (1 + S1 + S1 * S2)
CHUNK = 128
KCH = 40
RPW = CHUNK * KCH
GT = RPW * NW

BLK = 1000
NB = B // BLK

def _build_oidx() -> np.ndarray:
    p = np.arange(GT, dtype=np.int64)
    out = p.copy()
    s1lo, s1hi = B, B + B * S1
    q = p - s1lo
    sel = (p >= s1lo) & (p < s1hi)
    out[sel] = s1lo + (q[sel] % S1) * B + q[sel] // S1
    s2lo, s2hi = s1hi, s1hi + B * S1 * S2
    q = p - s2lo
    sel = (p >= s2lo) & (p < s2hi)
    out[sel] = s2lo + (q[sel] % (S1 * S2)) * B + q[sel] // (S1 * S2)
    return out.reshape(NW, KCH, CHUNK).astype(np.int32)


_OIDX = _build_oidx()
_PADIDX = np.tile(np.arange(GT - ROWS_T, dtype=np.int32)[None, :], (T, 1))


def _sc_gather_build(t: int):
    mesh = plsc.VectorSubcoreMesh(core_axis_name="c", subcore_axis_name="s")
    tn = t * N

    @functools.partial(
        pl.kernel,
        out_type=jax.ShapeDtypeStruct((GT, D), jnp.float32),
        mesh=mesh,
        scratch_types=[
            pltpu.VMEM((RPW,), jnp.int32),
            pltpu.VMEM((KCH, CHUNK), jnp.int32),
            pltpu.VMEM((CHUNK, D), jnp.float32),
            pltpu.SemaphoreType.DMA,
            pltpu.SemaphoreType.DMA,
        ],
    )
    def sc_gather(table, idx, oidx, out, iv, ov, rv, gsem, wsem):
        wid = lax.axis_index("s") * NC + lax.axis_index("c")
        base_w = t * GT + wid * RPW
        pltpu.sync_copy(idx.at[pl.ds(base_w, RPW)], iv)
        pltpu.sync_copy(oidx.at[wid], ov)

        if tn:
            def shift(i, carry):
                iv[pl.ds(i * 16, 16)] = iv[pl.ds(i * 16, 16)] + tn
                return carry

            lax.fori_loop(0, RPW // 16, shift, 0)

        def step(i, carry):
            pltpu.async_copy(table.at[iv.at[pl.ds(i * CHUNK, CHUNK)]],
                             rv, gsem).wait()
            pltpu.async_copy(rv, out.at[ov.at[i]], wsem).wait()
            return carry

        lax.fori_loop(0, KCH, step, 0)

    return sc_gather


_sc_gather_t = [_sc_gather_build(t) for t in range(T)]


def _tc_body(h0, h10, h11, h12, h13, h14,
             h20, h21, h22, h23, h24, h25, h26, h27, h28, h29,
             w1, w2, wp, out):
    h1b = (h10, h11, h12, h13, h14)
    h2b = (h20, h21, h22, h23, h24, h25, h26, h27, h28, h29)
    w1v = w1[...]
    n0 = h1b[0][...]
    for k in range(1, S1):
        n0 = n0 + h1b[k][...]
    a0 = h0[...] + n0 * (1.0 / S1)
    g = (jnp.dot(a0, w1v, preferred_element_type=jnp.float32)
         >= VTH).astype(jnp.float32)
    gs = jnp.zeros((BLK, H1), jnp.float32)
    for k in range(S1):
        a1 = h1b[k][...] + 0.5 * (h2b[2 * k][...] + h2b[2 * k + 1][...])
        s1 = (jnp.dot(a1, w1v, preferred_element_type=jnp.float32)
              >= VTH).astype(jnp.float32)
        gs = gs + s1
    g2 = g + gs * (1.0 / S1)
    o2 = jnp.dot(g2, w2[...], preferred_element_type=jnp.float32)
    s2 = (o2 >= VTH).astype(jnp.float32)
    out[...] = jnp.dot(s2, wp[0], preferred_element_type=jnp.float32)


def _tc_net_build(t: int):
    in_specs = (
        [pl.BlockSpec((BLK, D), lambda i: (i, 0))]
        + [pl.BlockSpec((BLK, D), lambda i, k=k: ((1 + k) * NB + i, 0))
           for k in range(S1)]
        + [pl.BlockSpec((BLK, D),
                        lambda i, m=m: ((1 + S1 + m) * NB + i, 0))
           for m in range(S1 * S2)]
        + [
            pl.BlockSpec((D, H1), lambda i: (0, 0)),
            pl.BlockSpec((H1, H2), lambda i: (0, 0)),
            pl.BlockSpec((1, H2, OUT), lambda i, t=t: (t, 0, 0)),
        ]
    )
    return pl.pallas_call(
        _tc_body,
        grid=(NB,),
        in_specs=in_specs,
        out_specs=pl.BlockSpec((BLK, OUT), lambda i: (i, 0)),
        out_shape=jax.ShapeDtypeStruct((B, OUT), jnp.float32),
    )


_tc_net_t = [_tc_net_build(t) for t in range(T)]


def kernel(x, nodes, nbr1, nbr2, W1, W2, Wp, bp):
    table = x.reshape(T * N, D)
    idx = jnp.concatenate(
        [jnp.broadcast_to(nodes.astype(jnp.int32)[None, :], (T, B)),
         nbr1.astype(jnp.int32),
         nbr2.astype(jnp.int32),
         jnp.asarray(_PADIDX)],
        axis=1).reshape(-1)
    oidx = jnp.asarray(_OIDX)
    wp3 = Wp.reshape(T, H2, OUT)

    acc = None
    for t in range(T):
        hg = _sc_gather_t[t](table, idx, oidx)
        args = [hg] * 16 + [W1, W2, wp3]
        c = _tc_net_t[t](*args)
        acc = c if acc is None else acc + c
    return acc + bp[None, :]


# speedup vs baseline: 7.9484x; 1.1549x over previous
import functools

import numpy as np

import jax
import jax.numpy as jnp
from jax import lax
from jax.experimental import pallas as pl
from jax.experimental.pallas import tpu as pltpu
from jax.experimental.pallas import tpu_sc as plsc

T = 3
N = 100000
D = 128
B = 10000
S1, S2 = 5, 2
H1, H2 = 512, 10
OUT = 32
VTH = 1.0

NC, NS = 2, 16
NW = NC * NS
ROWS_T = B * (1 + S1 + S1 * S2)
CHUNK = 128
KCH = 40
RPW = CHUNK * KCH
GT = RPW * NW

BLK = 1000
NB = B // BLK

def _build_oidx() -> np.ndarray:
    p = np.arange(GT, dtype=np.int64)
    out = p.copy()
    s1lo, s1hi = B, B + B * S1
    q = p - s1lo
    sel = (p >= s1lo) & (p < s1hi)
    out[sel] = s1lo + (q[sel] % S1) * B + q[sel] // S1
    s2lo, s2hi = s1hi, s1hi + B * S1 * S2
    q = p - s2lo
    sel = (p >= s2lo) & (p < s2hi)
    out[sel] = s2lo + (q[sel] % (S1 * S2)) * B + q[sel] // (S1 * S2)
    return out.reshape(NW, KCH, CHUNK).astype(np.int32)


_OIDX = _build_oidx()
_PADIDX = np.arange(GT - ROWS_T, dtype=np.int32)


def _sc_gather_build():
    mesh = plsc.VectorSubcoreMesh(core_axis_name="c", subcore_axis_name="s")

    @functools.partial(
        pl.kernel,
        out_type=jax.ShapeDtypeStruct((GT, D), jnp.float32),
        mesh=mesh,
        scratch_types=[
            pltpu.VMEM((RPW,), jnp.int32),
            pltpu.VMEM((KCH, CHUNK), jnp.int32),
            pltpu.VMEM((CHUNK, D), jnp.float32),
            pltpu.VMEM((CHUNK, D), jnp.float32),
            pltpu.SemaphoreType.DMA,
            pltpu.SemaphoreType.DMA,
            pltpu.SemaphoreType.DMA,
            pltpu.SemaphoreType.DMA,
        ],
    )
    def sc_gather(table, idx, oidx, out, iv, ov, rva, rvb, ga, gb, wa, wb):
        wid = lax.axis_index("s") * NC + lax.axis_index("c")
        pltpu.sync_copy(idx.at[pl.ds(wid * RPW, RPW)], iv)
        pltpu.sync_copy(oidx.at[wid], ov)

        def gstart(c, rv, sem):
            pltpu.make_async_copy(
                table.at[iv.at[pl.ds(c * CHUNK, CHUNK)]], rv, sem).start()

        def gwait(rv, sem):
            pltpu.make_async_copy(
                table.at[iv.at[pl.ds(0, CHUNK)]], rv, sem).wait()

        def wstart(c, rv, sem):
            pltpu.make_async_copy(rv, out.at[ov.at[c]], sem).start()

        def wwait(rv, sem):
            pltpu.make_async_copy(rv, out.at[ov.at[0]], sem).wait()

        gstart(0, rva, ga)
        gstart(1, rvb, gb)
        gwait(rva, ga)
        wstart(0, rva, wa)
        gwait(rvb, gb)
        wstart(1, rvb, wb)
        wwait(rva, wa)
        gstart(2, rva, ga)

        def body(j, carry):
            c0 = 2 * j
            gwait(rva, ga)
            wstart(c0, rva, wa)
            wwait(rvb, wb)
            gstart(c0 + 1, rvb, gb)
            gwait(rvb, gb)
            wstart(c0 + 1, rvb, wb)
            wwait(rva, wa)

            @pl.when(c0 + 2 < KCH)
            def _():
                gstart(c0 + 2, rva, ga)

            return carry

        lax.fori_loop(1, KCH // 2, body, 0)
        wwait(rvb, wb)

    return sc_gather


_sc_gather = _sc_gather_build()


def _tc_body_build(t: int):
    def body(h0, h10, h11, h12, h13, h14,
             h20, h21, h22, h23, h24, h25, h26, h27, h28, h29,
             w1, w2, wp, prev, out):
        h1b = (h10, h11, h12, h13, h14)
        h2b = (h20, h21, h22, h23, h24, h25, h26, h27, h28, h29)
        w1v = w1[...]
        n0 = h1b[0][...]
        for k in range(1, S1):
            n0 = n0 + h1b[k][...]
        a0 = h0[...] + n0 * (1.0 / S1)
        g = (jnp.dot(a0, w1v, preferred_element_type=jnp.float32)
             >= VTH).astype(jnp.float32)
        gs = jnp.zeros((BLK, H1), jnp.float32)
        for k in range(S1):
            a1 = h1b[k][...] + 0.5 * (h2b[2 * k][...] + h2b[2 * k + 1][...])
            s1 = (jnp.dot(a1, w1v, preferred_element_type=jnp.float32)
                  >= VTH).astype(jnp.float32)
            gs = gs + s1
        g2 = g + gs * (1.0 / S1)
        o2 = jnp.dot(g2, w2[...], preferred_element_type=jnp.float32)
        s2 = (o2 >= VTH).astype(jnp.float32)
        out[...] = prev[...] + jnp.dot(s2, wp[0],
                                       preferred_element_type=jnp.float32)

    return body


def _tc_net_build(t: int):
    prev_spec = (pl.BlockSpec((1, OUT), lambda i: (0, 0)) if t == 0
                 else pl.BlockSpec((BLK, OUT), lambda i: (i, 0)))
    in_specs = (
        [pl.BlockSpec((BLK, D), lambda i: (i, 0))]
        + [pl.BlockSpec((BLK, D), lambda i, k=k: ((1 + k) * NB + i, 0))
           for k in range(S1)]
        + [pl.BlockSpec((BLK, D),
                        lambda i, m=m: ((1 + S1 + m) * NB + i, 0))
           for m in range(S1 * S2)]
        + [
            pl.BlockSpec((D, H1), lambda i: (0, 0)),
            pl.BlockSpec((H1, H2), lambda i: (0, 0)),
            pl.BlockSpec((1, H2, OUT), lambda i, t=t: (t, 0, 0)),
            prev_spec,
        ]
    )
    return pl.pallas_call(
        _tc_body_build(t),
        grid=(NB,),
        in_specs=in_specs,
        out_specs=pl.BlockSpec((BLK, OUT), lambda i: (i, 0)),
        out_shape=jax.ShapeDtypeStruct((B, OUT), jnp.float32),
    )


_tc_net_t = [_tc_net_build(t) for t in range(T)]


def kernel(x, nodes, nbr1, nbr2, W1, W2, Wp, bp):
    table = x.reshape(T * N, D)
    nodes_i = nodes.astype(jnp.int32)
    oidx = jnp.asarray(_OIDX)
    pad = jnp.asarray(_PADIDX)
    wp3 = Wp.reshape(T, H2, OUT)

    prev = bp.reshape(1, OUT)
    for t in range(T):
        idx_t = jnp.concatenate(
            [nodes_i, nbr1[t].astype(jnp.int32),
             nbr2[t].astype(jnp.int32), pad]) + (t * N)
        hg = _sc_gather(table, idx_t, oidx)
        args = [hg] * 16 + [W1, W2, wp3, prev]
        prev = _tc_net_t[t](*args)
    return prev


# speedup vs baseline: 8.6765x; 1.0916x over previous
import functools

import numpy as np

import jax
import jax.numpy as jnp
from jax import lax
from jax.experimental import pallas as pl
from jax.experimental.pallas import tpu as pltpu
from jax.experimental.pallas import tpu_sc as plsc

T = 3
N = 100000
D = 128
B = 10000
S1, S2 = 5, 2
H1, H2 = 512, 10
OUT = 32
VTH = 1.0

NC, NS = 2, 16
NW = NC * NS
ROWS_T = B * (1 + S1 + S1 * S2)
CHUNK = 128
KCH = 40
RPW = CHUNK * KCH
GT = RPW * NW

BLK = 1000
NB = B // BLK

def _build_oidx() -> np.ndarray:
    p = np.arange(GT, dtype=np.int64)
    out = p.copy()
    s1lo, s1hi = B, B + B * S1
    q = p - s1lo
    sel = (p >= s1lo) & (p < s1hi)
    out[sel] = s1lo + (q[sel] % S1) * B + q[sel] // S1
    s2lo, s2hi = s1hi, s1hi + B * S1 * S2
    q = p - s2lo
    sel = (p >= s2lo) & (p < s2hi)
    out[sel] = s2lo + (q[sel] % (S1 * S2)) * B + q[sel] // (S1 * S2)
    return out.reshape(NW, KCH, CHUNK).astype(np.int32)


_OIDX = _build_oidx()
_PADIDX = np.arange(GT - ROWS_T, dtype=np.int32)


def _sc_gather_build():
    mesh = plsc.VectorSubcoreMesh(core_axis_name="c", subcore_axis_name="s")

    @functools.partial(
        pl.kernel,
        out_type=jax.ShapeDtypeStruct((GT, D), jnp.float32),
        mesh=mesh,
        scratch_types=[
            pltpu.VMEM((RPW,), jnp.int32),
            pltpu.VMEM((KCH, CHUNK), jnp.int32),
            pltpu.VMEM((4, CHUNK, D), jnp.float32),
            pltpu.SemaphoreType.DMA,
            pltpu.SemaphoreType.DMA,
            pltpu.SemaphoreType.DMA,
            pltpu.SemaphoreType.DMA,
            pltpu.SemaphoreType.DMA,
            pltpu.SemaphoreType.DMA,
            pltpu.SemaphoreType.DMA,
            pltpu.SemaphoreType.DMA,
        ],
    )
    def sc_gather(table, idx, oidx, out, iv, ov, rv,
                  g0, g1, g2, g3, w0, w1, w2, w3):
        wid = lax.axis_index("s") * NC + lax.axis_index("c")
        pltpu.sync_copy(idx.at[pl.ds(wid * RPW, RPW)], iv)
        pltpu.sync_copy(oidx.at[wid], ov)
        gsem = (g0, g1, g2, g3)
        wsem = (w0, w1, w2, w3)

        def gstart(c, s):
            pltpu.make_async_copy(
                table.at[iv.at[pl.ds(c * CHUNK, CHUNK)]],
                rv.at[s], gsem[s]).start()

        def gwait(s):
            pltpu.make_async_copy(
                table.at[iv.at[pl.ds(0, CHUNK)]], rv.at[s], gsem[s]).wait()

        def wstart(c, s):
            pltpu.make_async_copy(rv.at[s], out.at[ov.at[c]],
                                  wsem[s]).start()

        def wwait(s):
            pltpu.make_async_copy(rv.at[0], out.at[ov.at[0]],
                                  wsem[s]).wait()

        gstart(0, 0)
        gstart(1, 1)
        for s in range(4):
            c = s
            gwait(s)
            wstart(c, s)
            if c >= 2:
                wwait((c + 2) % 4)
            gstart(c + 2, (s + 2) % 4)

        def body(j, carry):
            for s in range(4):
                c = 4 * j + s
                gwait(s)
                wstart(c, s)
                wwait((s + 2) % 4)
                gstart(c + 2, (s + 2) % 4)
            return carry

        lax.fori_loop(1, KCH // 4 - 1, body, 0)
        for s in range(4):
            c = KCH - 4 + s
            gwait(s)
            wstart(c, s)
            wwait((s + 2) % 4)
            if c + 2 < KCH:
                gstart(c + 2, (s + 2) % 4)
        wwait(2)
        wwait(3)

    return sc_gather


_sc_gather = _sc_gather_build()


def _tc_body_build(t: int):
    def body(h0, h10, h11, h12, h13, h14,
             h20, h21, h22, h23, h24, h25, h26, h27, h28, h29,
             w1, w2, wp, prev, out):
        h1b = (h10, h11, h12, h13, h14)
        h2b = (h20, h21, h22, h23, h24, h25, h26, h27, h28, h29)
        w1v = w1[...]
        n0 = h1b[0][...]
        for k in range(1, S1):
            n0 = n0 + h1b[k][...]
        a0 = h0[...] + n0 * (1.0 / S1)
        g = (jnp.dot(a0, w1v, preferred_element_type=jnp.float32)
             >= VTH).astype(jnp.float32)
        gs = jnp.zeros((BLK, H1), jnp.float32)
        for k in range(S1):
            a1 = h1b[k][...] + 0.5 * (h2b[2 * k][...] + h2b[2 * k + 1][...])
            s1 = (jnp.dot(a1, w1v, preferred_element_type=jnp.float32)
                  >= VTH).astype(jnp.float32)
            gs = gs + s1
        g2 = g + gs * (1.0 / S1)
        o2 = jnp.dot(g2, w2[...], preferred_element_type=jnp.float32)
        s2 = (o2 >= VTH).astype(jnp.float32)
        out[...] = prev[...] + jnp.dot(s2, wp[0],
                                       preferred_element_type=jnp.float32)

    return body


def _tc_net_build(t: int):
    prev_spec = (pl.BlockSpec((1, OUT), lambda i: (0, 0)) if t == 0
                 else pl.BlockSpec((BLK, OUT), lambda i: (i, 0)))
    in_specs = (
        [pl.BlockSpec((BLK, D), lambda i: (i, 0))]
        + [pl.BlockSpec((BLK, D), lambda i, k=k: ((1 + k) * NB + i, 0))
           for k in range(S1)]
        + [pl.BlockSpec((BLK, D),
                        lambda i, m=m: ((1 + S1 + m) * NB + i, 0))
           for m in range(S1 * S2)]
        + [
            pl.BlockSpec((D, H1), lambda i: (0, 0)),
            pl.BlockSpec((H1, H2), lambda i: (0, 0)),
            pl.BlockSpec((1, H2, OUT), lambda i, t=t: (t, 0, 0)),
            prev_spec,
        ]
    )
    return pl.pallas_call(
        _tc_body_build(t),
        grid=(NB,),
        in_specs=in_specs,
        out_specs=pl.BlockSpec((BLK, OUT), lambda i: (i, 0)),
        out_shape=jax.ShapeDtypeStruct((B, OUT), jnp.float32),
    )


_tc_net_t = [_tc_net_build(t) for t in range(T)]


def kernel(x, nodes, nbr1, nbr2, W1, W2, Wp, bp):
    table = x.reshape(T * N, D)
    nodes_i = nodes.astype(jnp.int32)
    oidx = jnp.asarray(_OIDX)
    pad = jnp.asarray(_PADIDX)
    wp3 = Wp.reshape(T, H2, OUT)

    prev = bp.reshape(1, OUT)
    for t in range(T):
        idx_t = jnp.concatenate(
            [nodes_i, nbr1[t].astype(jnp.int32),
             nbr2[t].astype(jnp.int32), pad]) + (t * N)
        hg = _sc_gather(table, idx_t, oidx)
        args = [hg] * 16 + [W1, W2, wp3, prev]
        prev = _tc_net_t[t](*args)
    return prev
